# Initial kernel scaffold; baseline (speedup 1.0000x reference)
import jax
import jax.numpy as jnp
from jax.experimental import pallas as pl


def kernel(batch, positions, external_potential):
    raise NotImplementedError("write your pallas kernel here")


# speedup vs baseline: 1.9898x; 1.9898x over previous
import functools

import jax
import jax.numpy as jnp
from jax import lax
from jax.experimental import pallas as pl
from jax.experimental.pallas import tpu as pltpu
from jax.experimental.pallas import tpu_sc as plsc

N_NODES = 100000
N_GRAPHS = 512
D_OUT = 36
CHUNK = 512
SUB = 128
NW = 32
N_CHUNKS = -(-N_NODES // CHUNK)
MAX_ITERS = -(-N_CHUNKS // NW)


def _table_body(ep_ref, out_ref):
    out_ref[...] = jnp.zeros((N_GRAPHS, D_OUT), jnp.float32)
    ep = ep_ref[...]
    for w in range(4):
        out_ref[:, 9 * w:9 * w + 4] = ep


def _build_table(ep):
    return pl.pallas_call(
        _table_body,
        out_shape=jax.ShapeDtypeStruct((N_GRAPHS, D_OUT), jnp.float32),
    )(ep)


def _gather_body(batch_hbm, table_hbm, out_hbm, idx_v, rows_v, sem):
    wid = lax.axis_index("s") * 2 + lax.axis_index("c")
    for i in range(MAX_ITERS):
        c = wid + NW * i
        @pl.when(c < N_CHUNKS)
        def _():
            base = jnp.minimum(c * CHUNK, N_NODES - CHUNK)
            for j in range(CHUNK // SUB):
                pltpu.sync_copy(batch_hbm.at[pl.ds(base + j * SUB, SUB)],
                                idx_v.at[j])
            copies = []
            for j in range(CHUNK // SUB):
                copies.append(pltpu.async_copy(
                    table_hbm.at[idx_v.at[j]],
                    rows_v.at[pl.ds(j * SUB, SUB), :], sem))
            for cp in copies:
                cp.wait()
            pltpu.sync_copy(rows_v, out_hbm.at[pl.ds(base, CHUNK), :])


@functools.partial(jax.jit, static_argnames=())
def _gather(batch, table):
    mesh = plsc.VectorSubcoreMesh(core_axis_name="c", subcore_axis_name="s")
    return pl.kernel(
        _gather_body,
        out_type=jax.ShapeDtypeStruct((N_NODES, D_OUT), jnp.float32),
        mesh=mesh,
        scratch_types=[
            pltpu.VMEM((CHUNK // SUB, SUB), jnp.int32),
            pltpu.VMEM((CHUNK, D_OUT), jnp.float32),
            pltpu.SemaphoreType.DMA,
        ],
        compiler_params=pltpu.CompilerParams(use_tc_tiling_on_sc=False),
    )(batch, table)


def kernel(batch, positions, external_potential):
    table = _build_table(external_potential.astype(jnp.float32))
    return _gather(batch.astype(jnp.int32), table)
